# Initial kernel scaffold; baseline (speedup 1.0000x reference)
import functools

import jax
import jax.numpy as jnp
from jax import lax
from jax.experimental import pallas as pl
from jax.experimental.pallas import tpu as pltpu
from jax.experimental.pallas import tpu_sc as plsc

_N = 10000
_D = 128
_E = 320000
_NC, _NS = 2, 16
_NW = _NC * _NS
_CH = 128
_EPT = 10240
_NCHUNK = _EPT // _CH
_ACC = 10240
_RPT = _ACC // _NS


def _mm_body(x_ref, w_ref, o_ref):
    o_ref[...] = jnp.dot(x_ref[...], w_ref[...],
                         preferred_element_type=jnp.float32)


def _matmul(x, W):
    return pl.pallas_call(
        _mm_body,
        grid=(10,),
        in_specs=[
            pl.BlockSpec((_N // 10, _D), lambda i: (i, 0)),
            pl.BlockSpec((_D, _D), lambda i: (0, 0)),
        ],
        out_specs=pl.BlockSpec((_N // 10, _D), lambda i: (i, 0)),
        out_shape=jax.ShapeDtypeStruct((_N, _D), jnp.float32),
    )(x, W)


_mesh = plsc.VectorSubcoreMesh(core_axis_name="c", subcore_axis_name="s")


@functools.partial(
    pl.kernel,
    mesh=_mesh,
    out_type=jax.ShapeDtypeStruct((_NC, _ACC, _D), jnp.float32),
    scratch_types=[
        pltpu.VMEM((_NCHUNK, _CH), jnp.int32),
        pltpu.VMEM((_NCHUNK, _CH), jnp.int32),
        pltpu.VMEM((_CH, _D), jnp.float32),
        pltpu.VMEM((_CH, _D), jnp.float32),
        pltpu.VMEM_SHARED((_ACC, _D), jnp.float32),
        pltpu.SemaphoreType.DMA,
        pltpu.SemaphoreType.DMA,
    ],
)
def _sc_segsum(h_hbm, src_hbm, dst_hbm, out_hbm,
               src_v, dst_v, buf0, buf1, acc, sem0, sem1):
    c = lax.axis_index("c")
    s = lax.axis_index("s")
    wid = c * _NS + s

    pltpu.sync_copy(src_hbm.at[wid], src_v)
    pltpu.sync_copy(dst_hbm.at[wid], dst_v)

    def _zbody(i, carry):
        for k in range(_D // 16):
            buf0[i, pl.ds(k * 16, 16)] = jnp.zeros((16,), jnp.float32)
        return carry
    lax.fori_loop(0, _CH, _zbody, 0)
    for k in range(_RPT // _CH):
        pltpu.sync_copy(buf0, acc.at[pl.ds(s * _RPT + k * _CH, _CH)])
    plsc.subcore_barrier()

    def _body(jp, carry):
        j = jp * 2
        pltpu.make_async_copy(h_hbm.at[src_v.at[j]], buf0, sem0).wait()
        pltpu.async_copy(h_hbm.at[src_v.at[j + 1]], buf1, sem1)
        pltpu.sync_copy(buf0, acc.at[dst_v.at[j]], add=True)
        pltpu.make_async_copy(h_hbm.at[src_v.at[j + 1]], buf1, sem1).wait()
        jn = lax.rem(j + 2, _NCHUNK)
        pltpu.async_copy(h_hbm.at[src_v.at[jn]], buf0, sem0)
        pltpu.sync_copy(buf1, acc.at[dst_v.at[j + 1]], add=True)
        return carry

    pltpu.async_copy(h_hbm.at[src_v.at[0]], buf0, sem0)
    lax.fori_loop(0, _NCHUNK // 2, _body, 0)
    pltpu.make_async_copy(h_hbm.at[src_v.at[0]], buf0, sem0).wait()

    plsc.subcore_barrier()
    pltpu.sync_copy(acc.at[pl.ds(s * _RPT, _RPT)],
                    out_hbm.at[c, pl.ds(s * _RPT, _RPT)])


def _comb_body(p_ref, b_ref, o_ref):
    o_ref[...] = jnp.maximum(p_ref[0] + p_ref[1] + b_ref[...], 0.0)


def _combine(p, b):
    blk = _N // 10
    return pl.pallas_call(
        _comb_body,
        grid=(10,),
        in_specs=[
            pl.BlockSpec((2, blk, _D), lambda i: (0, i, 0)),
            pl.BlockSpec((1, _D), lambda i: (0, 0)),
        ],
        out_specs=pl.BlockSpec((blk, _D), lambda i: (i, 0)),
        out_shape=jax.ShapeDtypeStruct((_N, _D), jnp.float32),
    )(p, b.reshape(1, _D))


def kernel(x, adj_t, W, b):
    h = _matmul(x, W)
    src = adj_t[0]
    dst = adj_t[1]
    pad = _NW * _EPT - _E
    srcp = jnp.concatenate([src, jnp.zeros((pad,), jnp.int32)])
    dstp = jnp.concatenate([dst, jnp.full((pad,), _N, jnp.int32)])
    srcp = srcp.reshape(_NW, _NCHUNK, _CH)
    dstp = dstp.reshape(_NW, _NCHUNK, _CH)
    partials = _sc_segsum(h, srcp, dstp)
    return _combine(partials, b)


# speedup vs baseline: 2.3156x; 2.3156x over previous
import functools

import jax
import jax.numpy as jnp
from jax import lax
from jax.experimental import pallas as pl
from jax.experimental.pallas import tpu as pltpu
from jax.experimental.pallas import tpu_sc as plsc

_N = 10000
_D = 128
_E = 320000
_NC, _NS = 2, 16
_CH = 128
_NCHUNK = 160
_EPT = _NCHUNK * _CH
_EPAD = _NS * _EPT
_HN = _N // 2
_ACC = 5120
_RPT = _ACC // _NS


def _mm_body(x_ref, w_ref, o_ref):
    o_ref[...] = jnp.dot(x_ref[...], w_ref[...],
                         preferred_element_type=jnp.float32)


def _matmul(x, W):
    blk = _N // 10
    return pl.pallas_call(
        _mm_body,
        grid=(10,),
        in_specs=[
            pl.BlockSpec((blk, _D), lambda i: (i, 0)),
            pl.BlockSpec((_D, _D), lambda i: (0, 0)),
        ],
        out_specs=pl.BlockSpec((blk, _D), lambda i: (i, 0)),
        out_shape=jax.ShapeDtypeStruct((_N, _D), jnp.float32),
    )(x, W)


_mesh = plsc.VectorSubcoreMesh(core_axis_name="c", subcore_axis_name="s")


@functools.partial(
    pl.kernel,
    mesh=_mesh,
    out_type=jax.ShapeDtypeStruct((_NC, _ACC, _D), jnp.float32),
    scratch_types=[
        pltpu.VMEM((_NCHUNK, _CH), jnp.int32),
        pltpu.VMEM((_NCHUNK, _CH), jnp.int32),
        pltpu.VMEM((_CH, _D), jnp.float32),
        pltpu.VMEM((_CH, _D), jnp.float32),
        pltpu.VMEM_SHARED((_ACC, _D), jnp.float32),
        pltpu.SemaphoreType.DMA,
        pltpu.SemaphoreType.DMA,
    ],
)
def _sc_segsum(h_hbm, src_hbm, dst_hbm, out_hbm,
               src_v, dst_v, buf0, buf1, acc, sem0, sem1):
    c = lax.axis_index("c")
    s = lax.axis_index("s")

    pltpu.sync_copy(src_hbm.at[s], src_v)
    pltpu.sync_copy(dst_hbm.at[s], dst_v)

    base = c * _HN

    def _rbody(i, carry):
        for k in range(_D // 16):
            d = dst_v[i, pl.ds(k * 16, 16)] - base
            ok = (d >= 0) & (d < _HN)
            dst_v[i, pl.ds(k * 16, 16)] = jnp.where(ok, d, _HN)
        return carry
    lax.fori_loop(0, _NCHUNK, _rbody, 0)

    def _zbody(i, carry):
        for k in range(_D // 16):
            buf0[i, pl.ds(k * 16, 16)] = jnp.zeros((16,), jnp.float32)
        return carry
    lax.fori_loop(0, _CH, _zbody, 0)
    for k in range((_RPT + _CH - 1) // _CH):
        nrow = min(_CH, _RPT - k * _CH)
        pltpu.sync_copy(buf0.at[pl.ds(0, nrow)],
                        acc.at[pl.ds(s * _RPT + k * _CH, nrow)])
    plsc.subcore_barrier()

    def _body(jp, carry):
        j = jp * 2
        pltpu.make_async_copy(h_hbm.at[src_v.at[j]], buf0, sem0).wait()
        pltpu.async_copy(h_hbm.at[src_v.at[j + 1]], buf1, sem1)
        pltpu.sync_copy(buf0, acc.at[dst_v.at[j]], add=True)
        pltpu.make_async_copy(h_hbm.at[src_v.at[j + 1]], buf1, sem1).wait()
        jn = lax.rem(j + 2, _NCHUNK)
        pltpu.async_copy(h_hbm.at[src_v.at[jn]], buf0, sem0)
        pltpu.sync_copy(buf1, acc.at[dst_v.at[j + 1]], add=True)
        return carry

    pltpu.async_copy(h_hbm.at[src_v.at[0]], buf0, sem0)
    lax.fori_loop(0, _NCHUNK // 2, _body, 0)
    pltpu.make_async_copy(h_hbm.at[src_v.at[0]], buf0, sem0).wait()

    plsc.subcore_barrier()
    pltpu.sync_copy(acc.at[pl.ds(s * _RPT, _RPT)],
                    out_hbm.at[c, pl.ds(s * _RPT, _RPT)])


def _comb_body(p_ref, b_ref, o_ref):
    o_ref[...] = jnp.maximum(p_ref[0] + b_ref[...], 0.0)


def _combine(p, b):
    blk = _N // 10
    nb = _HN // blk
    return pl.pallas_call(
        _comb_body,
        grid=(10,),
        in_specs=[
            pl.BlockSpec((1, blk, _D), lambda i: (i // nb, i % nb, 0)),
            pl.BlockSpec((1, _D), lambda i: (0, 0)),
        ],
        out_specs=pl.BlockSpec((blk, _D), lambda i: (i, 0)),
        out_shape=jax.ShapeDtypeStruct((_N, _D), jnp.float32),
    )(p, b.reshape(1, _D))


def kernel(x, adj_t, W, b):
    h = _matmul(x, W)
    src = adj_t[0]
    dst = adj_t[1]
    pad = _EPAD - _E
    srcp = jnp.concatenate([src, jnp.zeros((pad,), jnp.int32)])
    dstp = jnp.concatenate([dst, jnp.full((pad,), _N, jnp.int32)])
    srcp = srcp.reshape(_NS, _NCHUNK, _CH)
    dstp = dstp.reshape(_NS, _NCHUNK, _CH)
    p = _sc_segsum(h, srcp, dstp)
    return _combine(p, b)
